# baseline (speedup 1.0000x reference)
import jax
import jax.numpy as jnp
import numpy as np
from jax.experimental import pallas as pl
from jax.experimental.pallas import tpu as pltpu

FIRE_THRESH = 0.85
B, T, D, N = 2, 8192, 2048, 512
ROWS = 256
NB = (B * T) // ROWS
STRIP = 16
C1 = float(np.sqrt(2.0 / np.pi))
C2 = float(np.sqrt(2.0 / np.pi) * 0.044715)


def _gelu(v):
    z = v * (C1 + C2 * (v * v))
    h = 0.5 * v
    return h + h * jnp.tanh(z)


def _body(x_ref, buf_ref, depl_ref, maskf_ref, logk_ref, logdr_ref, logfl_ref,
          ptr_ref, out_ref, nbuf_ref, ndepl_ref, nmask_ref, gate_ref,
          sums_ref, mrow_ref, csem, rsem):
    p = pl.program_id(0)
    i = pl.program_id(1)

    @pl.when(jnp.logical_and(p == 0, i == 0))
    def _init():
        sums_ref[...] = jnp.zeros_like(sums_ref)

    @pl.when(p == 0)
    def _accum():
        for j in range(ROWS // STRIP):
            sums_ref[...] += _gelu(x_ref[pl.ds(j * STRIP, STRIP), :])

    @pl.when(jnp.logical_and(p == 0, i == 1))
    def _copy_buf():
        pltpu.make_async_copy(buf_ref, nbuf_ref, csem).start()

    @pl.when(jnp.logical_and(p == 0, i == 2))
    def _normalize_buf():
        pltpu.make_async_copy(buf_ref, nbuf_ref, csem).wait()
        buf = buf_ref[...]
        bn = jnp.sqrt(jnp.sum(buf * buf, axis=1, keepdims=True))
        buf_ref[...] = buf / jnp.maximum(bn, 1e-12)

    @pl.when(jnp.logical_and(p == 1, i == 0))
    def _state():
        k_gate = jnp.clip(jnp.exp(logk_ref[0, 0]), 0.1, 8.0)
        depl_rate = 0.1 + 0.8 * (1.0 / (1.0 + jnp.exp(-logdr_ref[0, 0])))
        floor_val = 0.5 * (1.0 / (1.0 + jnp.exp(-logfl_ref[0, 0])))
        ptr = ptr_ref[0, 0]

        m = jnp.sum(sums_ref[...], axis=0, keepdims=True) * (1.0 / (B * T))
        m_w = m / jnp.maximum(jnp.sqrt(jnp.sum(m * m)), 1e-12)
        sims = jax.lax.dot_general(
            buf_ref[...], m_w, (((1,), (1,)), ((), ())),
            preferred_element_type=jnp.float32)
        sims = sims.reshape(1, N)
        sims = jnp.where(maskf_ref[...] > 0.5, sims, -1.0)
        max_sim = jnp.max(sims)
        iota = jax.lax.broadcasted_iota(jnp.int32, (1, N), 1)
        nearest = jnp.min(jnp.where(sims == max_sim, iota, N))
        depl = depl_ref[...]
        depl_level = jnp.sum(jnp.where(iota == nearest, depl, 0.0))
        raw_gate = jnp.exp(-k_gate * (1.0 - depl_level))
        gate_ref[0, 0] = floor_val + (1.0 - floor_val) * raw_gate

        fire = max_sim > FIRE_THRESH
        nd = depl * jnp.where(
            jnp.logical_and(iota == nearest, fire), depl_rate, 1.0)
        ndepl_ref[...] = jnp.where(iota == ptr, 1.0, nd)
        nmask_ref[...] = jnp.where(iota == ptr, 1.0, maskf_ref[...])

        mrow_ref[...] = m_w
        pltpu.make_async_copy(
            mrow_ref, nbuf_ref.at[pl.ds(ptr, 1), :], rsem).start()

    @pl.when(p == 1)
    def _scale():
        g = gate_ref[0, 0]
        for j in range(ROWS // STRIP):
            sl = pl.ds(j * STRIP, STRIP)
            out_ref[sl, :] = _gelu(x_ref[sl, :]) * g

    @pl.when(jnp.logical_and(p == 1, i == NB - 1))
    def _finish_row():
        ptr = ptr_ref[0, 0]
        pltpu.make_async_copy(
            mrow_ref, nbuf_ref.at[pl.ds(ptr, 1), :], rsem).wait()


@jax.jit
def _run(x2d, buf, depl2d, maskf2d, logk, logdr, logfl, ptr2d):
    grid = (2, NB)
    out, nbuf, ndepl, nmaskf = pl.pallas_call(
        _body,
        grid=grid,
        in_specs=[
            pl.BlockSpec((ROWS, D), lambda p, i: (i, 0)),
            pl.BlockSpec((N, D), lambda p, i: (0, 0)),
            pl.BlockSpec((1, N), lambda p, i: (0, 0)),
            pl.BlockSpec((1, N), lambda p, i: (0, 0)),
            pl.BlockSpec(memory_space=pltpu.SMEM),
            pl.BlockSpec(memory_space=pltpu.SMEM),
            pl.BlockSpec(memory_space=pltpu.SMEM),
            pl.BlockSpec(memory_space=pltpu.SMEM),
        ],
        out_specs=[
            pl.BlockSpec((ROWS, D), lambda p, i: (jnp.where(p == 0, 0, i), 0)),
            pl.BlockSpec(memory_space=pl.ANY),
            pl.BlockSpec((1, N), lambda p, i: (0, 0)),
            pl.BlockSpec((1, N), lambda p, i: (0, 0)),
        ],
        out_shape=[
            jax.ShapeDtypeStruct((B * T, D), jnp.float32),
            jax.ShapeDtypeStruct((N, D), jnp.float32),
            jax.ShapeDtypeStruct((1, N), jnp.float32),
            jax.ShapeDtypeStruct((1, N), jnp.float32),
        ],
        scratch_shapes=[
            pltpu.SMEM((1, 1), jnp.float32),
            pltpu.VMEM((STRIP, D), jnp.float32),
            pltpu.VMEM((1, D), jnp.float32),
            pltpu.SemaphoreType.DMA,
            pltpu.SemaphoreType.DMA,
        ],
        compiler_params=pltpu.CompilerParams(
            dimension_semantics=("arbitrary", "arbitrary")),
    )(x2d, buf, depl2d, maskf2d, logk, logdr, logfl, ptr2d)
    return out, nbuf, ndepl, nmaskf


def kernel(x, buf, depl, mask, log_k, logit_depl_rate, logit_floor, ptr):
    x2d = x.reshape(B * T, D)
    depl2d = depl.reshape(1, N)
    maskf2d = mask.astype(jnp.float32).reshape(1, N)
    logk = log_k.reshape(1, 1)
    logdr = logit_depl_rate.reshape(1, 1)
    logfl = logit_floor.reshape(1, 1)
    ptr2d = ptr.reshape(1, 1)
    out, nbuf, ndepl, nmaskf = _run(
        x2d, buf, depl2d, maskf2d, logk, logdr, logfl, ptr2d)
    return (out.reshape(B, T, D), nbuf, ndepl.reshape(N),
            (nmaskf.reshape(N) > 0.5))


# speedup vs baseline: 1.3115x; 1.3115x over previous
import jax
import jax.numpy as jnp
import numpy as np
from jax.experimental import pallas as pl
from jax.experimental.pallas import tpu as pltpu

FIRE_THRESH = 0.85
B, T, D, N = 2, 8192, 2048, 512
ROWS = 1024
NB = (B * T) // ROWS
STRIP = 16
C1 = float(np.sqrt(2.0 / np.pi))
C2 = float(np.sqrt(2.0 / np.pi) * 0.044715)


def _gelu(v):
    z = v * (C1 + C2 * (v * v))
    h = 0.5 * v
    return h + h * jnp.tanh(z)


def _body(x_ref, buf_ref, depl_ref, maskf_ref, logk_ref, logdr_ref, logfl_ref,
          ptr_ref, out_ref, nbuf_ref, ndepl_ref, nmask_ref, gate_ref,
          sums_ref, mrow_ref, csem, rsem):
    p = pl.program_id(0)
    i = pl.program_id(1)

    @pl.when(jnp.logical_and(p == 0, i == 0))
    def _init():
        sums_ref[...] = jnp.zeros_like(sums_ref)

    @pl.when(p == 0)
    def _accum():
        for j in range(ROWS // STRIP):
            sums_ref[...] += _gelu(x_ref[pl.ds(j * STRIP, STRIP), :])

    @pl.when(jnp.logical_and(p == 0, i == 1))
    def _copy_buf():
        pltpu.make_async_copy(buf_ref, nbuf_ref, csem).start()

    @pl.when(jnp.logical_and(p == 0, i == 2))
    def _normalize_buf():
        pltpu.make_async_copy(buf_ref, nbuf_ref, csem).wait()
        buf = buf_ref[...]
        bn = jnp.sqrt(jnp.sum(buf * buf, axis=1, keepdims=True))
        buf_ref[...] = buf / jnp.maximum(bn, 1e-12)

    @pl.when(jnp.logical_and(p == 1, i == 0))
    def _state():
        k_gate = jnp.clip(jnp.exp(logk_ref[0, 0]), 0.1, 8.0)
        depl_rate = 0.1 + 0.8 * (1.0 / (1.0 + jnp.exp(-logdr_ref[0, 0])))
        floor_val = 0.5 * (1.0 / (1.0 + jnp.exp(-logfl_ref[0, 0])))
        ptr = ptr_ref[0, 0]

        m = jnp.sum(sums_ref[...], axis=0, keepdims=True) * (1.0 / (B * T))
        m_w = m / jnp.maximum(jnp.sqrt(jnp.sum(m * m)), 1e-12)
        sims = jax.lax.dot_general(
            buf_ref[...], m_w, (((1,), (1,)), ((), ())),
            preferred_element_type=jnp.float32)
        sims = sims.reshape(1, N)
        sims = jnp.where(maskf_ref[...] > 0.5, sims, -1.0)
        max_sim = jnp.max(sims)
        iota = jax.lax.broadcasted_iota(jnp.int32, (1, N), 1)
        nearest = jnp.min(jnp.where(sims == max_sim, iota, N))
        depl = depl_ref[...]
        depl_level = jnp.sum(jnp.where(iota == nearest, depl, 0.0))
        raw_gate = jnp.exp(-k_gate * (1.0 - depl_level))
        gate_ref[0, 0] = floor_val + (1.0 - floor_val) * raw_gate

        fire = max_sim > FIRE_THRESH
        nd = depl * jnp.where(
            jnp.logical_and(iota == nearest, fire), depl_rate, 1.0)
        ndepl_ref[...] = jnp.where(iota == ptr, 1.0, nd)
        nmask_ref[...] = jnp.where(iota == ptr, 1.0, maskf_ref[...])

        mrow_ref[...] = m_w
        pltpu.make_async_copy(
            mrow_ref, nbuf_ref.at[pl.ds(ptr, 1), :], rsem).start()

    @pl.when(p == 1)
    def _scale():
        g = gate_ref[0, 0]
        for j in range(ROWS // STRIP):
            sl = pl.ds(j * STRIP, STRIP)
            out_ref[sl, :] = _gelu(x_ref[sl, :]) * g

    @pl.when(jnp.logical_and(p == 1, i == NB - 1))
    def _finish_row():
        ptr = ptr_ref[0, 0]
        pltpu.make_async_copy(
            mrow_ref, nbuf_ref.at[pl.ds(ptr, 1), :], rsem).wait()


@jax.jit
def _run(x2d, buf, depl2d, maskf2d, logk, logdr, logfl, ptr2d):
    grid = (2, NB)
    out, nbuf, ndepl, nmaskf = pl.pallas_call(
        _body,
        grid=grid,
        in_specs=[
            pl.BlockSpec((ROWS, D), lambda p, i: (i, 0)),
            pl.BlockSpec((N, D), lambda p, i: (0, 0)),
            pl.BlockSpec((1, N), lambda p, i: (0, 0)),
            pl.BlockSpec((1, N), lambda p, i: (0, 0)),
            pl.BlockSpec(memory_space=pltpu.SMEM),
            pl.BlockSpec(memory_space=pltpu.SMEM),
            pl.BlockSpec(memory_space=pltpu.SMEM),
            pl.BlockSpec(memory_space=pltpu.SMEM),
        ],
        out_specs=[
            pl.BlockSpec((ROWS, D), lambda p, i: (jnp.where(p == 0, 0, i), 0)),
            pl.BlockSpec(memory_space=pl.ANY),
            pl.BlockSpec((1, N), lambda p, i: (0, 0)),
            pl.BlockSpec((1, N), lambda p, i: (0, 0)),
        ],
        out_shape=[
            jax.ShapeDtypeStruct((B * T, D), jnp.float32),
            jax.ShapeDtypeStruct((N, D), jnp.float32),
            jax.ShapeDtypeStruct((1, N), jnp.float32),
            jax.ShapeDtypeStruct((1, N), jnp.float32),
        ],
        scratch_shapes=[
            pltpu.SMEM((1, 1), jnp.float32),
            pltpu.VMEM((STRIP, D), jnp.float32),
            pltpu.VMEM((1, D), jnp.float32),
            pltpu.SemaphoreType.DMA,
            pltpu.SemaphoreType.DMA,
        ],
        compiler_params=pltpu.CompilerParams(
            dimension_semantics=("arbitrary", "arbitrary")),
    )(x2d, buf, depl2d, maskf2d, logk, logdr, logfl, ptr2d)
    return out, nbuf, ndepl, nmaskf


def kernel(x, buf, depl, mask, log_k, logit_depl_rate, logit_floor, ptr):
    x2d = x.reshape(B * T, D)
    depl2d = depl.reshape(1, N)
    maskf2d = mask.astype(jnp.float32).reshape(1, N)
    logk = log_k.reshape(1, 1)
    logdr = logit_depl_rate.reshape(1, 1)
    logfl = logit_floor.reshape(1, 1)
    ptr2d = ptr.reshape(1, 1)
    out, nbuf, ndepl, nmaskf = _run(
        x2d, buf, depl2d, maskf2d, logk, logdr, logfl, ptr2d)
    return (out.reshape(B, T, D), nbuf, ndepl.reshape(N),
            (nmaskf.reshape(N) > 0.5))
